# baseline (speedup 1.0000x reference)
import jax
import jax.numpy as jnp
from jax.experimental import pallas as pl
from jax.experimental.pallas import tpu as pltpu

N = 200000
C = 64
K = 20
VOXEL_SIZE = 0.02
BLOCK = 65536


def _fused_kernel(feat_ref, coord_ref, w1_ref, b1_ref, w2_ref, b2_ref,
                  wseg_ref, bseg_ref, pow_ref,
                  logit_ref, center_ref, seg_ref, mean_ref,
                  s_acc, c_acc):
    i = pl.program_id(0)

    feat = feat_ref[...]

    logits = jnp.dot(wseg_ref[...], feat, preferred_element_type=jnp.float32)
    logits = logits + bseg_ref[...]
    logit_ref[...] = logits

    h = jnp.dot(w1_ref[...], feat, preferred_element_type=jnp.float32)
    h = jnp.maximum(h + b1_ref[...], 0.0)
    bias = jnp.dot(w2_ref[...], h, preferred_element_type=jnp.float32)
    center_ref[...] = coord_ref[...] * (1.0 / VOXEL_SIZE) + (bias + b2_ref[...])

    colmax = jnp.max(logits, axis=0, keepdims=True)
    is_max = (logits == colmax).astype(jnp.float32)
    exps = jnp.exp(logits - colmax)
    packed = jnp.sum(is_max * pow_ref[...], axis=0, keepdims=True)
    expo = (jax.lax.bitcast_convert_type(packed, jnp.int32) >> 23) - 127
    idx = (K - 1) - expo
    seg_ref[...] = idx

    conf = 1.0 / jnp.sum(exps, axis=0, keepdims=True)
    col = jax.lax.broadcasted_iota(jnp.int32, idx.shape, 1) + i * BLOCK
    maskb = jnp.logical_and(idx >= 2, col < N)
    w = jnp.where(maskb, conf, 0.0)
    cnt = jnp.where(maskb, 1.0, 0.0)
    sub = jax.lax.broadcasted_iota(jnp.int32, logits.shape, 0)
    onehot = (sub == idx).astype(jnp.float32)
    scores = jnp.sum(onehot * w, axis=1, keepdims=True)
    counts = jnp.sum(onehot * cnt, axis=1, keepdims=True)

    prev_s = jnp.where(i == 0, jnp.zeros_like(scores), s_acc[...])
    prev_c = jnp.where(i == 0, jnp.zeros_like(counts), c_acc[...])
    s = prev_s + scores
    c = prev_c + counts
    s_acc[...] = s
    c_acc[...] = c
    mean_ref[...] = s / (c + 1e-8)


@jax.jit
def kernel(feat, coord, W1, b1, gamma, beta, rmean, rvar, W2, b2, Wseg, bseg):
    scale = gamma * jax.lax.rsqrt(rvar + 1e-3)
    w1t = W1.T * scale[:, None]
    b1t = ((b1 - rmean) * scale + beta)[:, None]
    w2t = W2.T * (1.0 / VOXEL_SIZE)
    b2t = (b2 * (1.0 / VOXEL_SIZE))[:, None]

    pw = jnp.asarray([float(1 << (K - 1 - j)) for j in range(K)],
                     jnp.float32)[:, None]

    grid = (N + BLOCK - 1) // BLOCK
    out_shape = (
        jax.ShapeDtypeStruct((K, N), jnp.float32),
        jax.ShapeDtypeStruct((3, N), jnp.float32),
        jax.ShapeDtypeStruct((1, N), jnp.int32),
        jax.ShapeDtypeStruct((K, 1), jnp.float32),
    )
    logit_t, center_t, seg_t, mean2d = pl.pallas_call(
        _fused_kernel,
        grid=(grid,),
        in_specs=[
            pl.BlockSpec((C, BLOCK), lambda i: (0, i)),
            pl.BlockSpec((3, BLOCK), lambda i: (0, i)),
            pl.BlockSpec((C, C), lambda i: (0, 0)),
            pl.BlockSpec((C, 1), lambda i: (0, 0)),
            pl.BlockSpec((3, C), lambda i: (0, 0)),
            pl.BlockSpec((3, 1), lambda i: (0, 0)),
            pl.BlockSpec((K, C), lambda i: (0, 0)),
            pl.BlockSpec((K, 1), lambda i: (0, 0)),
            pl.BlockSpec((K, 1), lambda i: (0, 0)),
        ],
        out_specs=[
            pl.BlockSpec((K, BLOCK), lambda i: (0, i)),
            pl.BlockSpec((3, BLOCK), lambda i: (0, i)),
            pl.BlockSpec((1, BLOCK), lambda i: (0, i)),
            pl.BlockSpec((K, 1), lambda i: (0, 0)),
        ],
        out_shape=out_shape,
        scratch_shapes=[
            pltpu.VMEM((K, 1), jnp.float32),
            pltpu.VMEM((K, 1), jnp.float32),
        ],
        compiler_params=pltpu.CompilerParams(
            dimension_semantics=("arbitrary",),
        ),
    )(feat.T, coord.T, w1t, b1t, w2t, b2t, Wseg.T, bseg[:, None], pw)

    return logit_t.T, center_t.T, seg_t[0], mean2d[:, 0]


# speedup vs baseline: 1.1247x; 1.1247x over previous
import jax
import jax.numpy as jnp
from jax.experimental import pallas as pl
from jax.experimental.pallas import tpu as pltpu

N = 200000
C = 64
K = 20
VOXEL_SIZE = 0.02
BLOCK = 24576


def _fused_kernel(feat_ref, coord_ref, w1_ref, b1_ref, w2_ref, b2_ref,
                  wseg_ref, bseg_ref, pow_ref,
                  logit_ref, center_ref, seg_ref, mean_ref,
                  s_acc, c_acc):
    i = pl.program_id(0)

    feat = feat_ref[...]

    logits = jnp.dot(wseg_ref[...], feat, preferred_element_type=jnp.float32)
    logits = logits + bseg_ref[...]
    logit_ref[...] = logits

    h = jnp.dot(w1_ref[...], feat, preferred_element_type=jnp.float32)
    h = jnp.maximum(h + b1_ref[...], 0.0)
    bias = jnp.dot(w2_ref[...], h, preferred_element_type=jnp.float32)
    center_ref[...] = coord_ref[...] * (1.0 / VOXEL_SIZE) + (bias + b2_ref[...])

    colmax = jnp.max(logits, axis=0, keepdims=True)
    is_max = (logits == colmax).astype(jnp.float32)
    exps = jnp.exp(logits - colmax)
    packed = jnp.sum(is_max * pow_ref[...], axis=0, keepdims=True)
    expo = (jax.lax.bitcast_convert_type(packed, jnp.int32) >> 23) - 127
    idx = (K - 1) - expo
    seg_ref[...] = idx

    conf = 1.0 / jnp.sum(exps, axis=0, keepdims=True)
    col = jax.lax.broadcasted_iota(jnp.int32, idx.shape, 1) + i * BLOCK
    maskb = jnp.logical_and(idx >= 2, col < N)
    w = jnp.where(maskb, conf, 0.0)
    cnt = jnp.where(maskb, 1.0, 0.0)
    sub = jax.lax.broadcasted_iota(jnp.int32, logits.shape, 0)
    onehot = (sub == idx).astype(jnp.float32)
    scores = jnp.sum(onehot * w, axis=1, keepdims=True)
    counts = jnp.sum(onehot * cnt, axis=1, keepdims=True)

    prev_s = jnp.where(i == 0, jnp.zeros_like(scores), s_acc[...])
    prev_c = jnp.where(i == 0, jnp.zeros_like(counts), c_acc[...])
    s = prev_s + scores
    c = prev_c + counts
    s_acc[...] = s
    c_acc[...] = c
    mean_ref[...] = s / (c + 1e-8)


@jax.jit
def kernel(feat, coord, W1, b1, gamma, beta, rmean, rvar, W2, b2, Wseg, bseg):
    scale = gamma * jax.lax.rsqrt(rvar + 1e-3)
    w1t = W1.T * scale[:, None]
    b1t = ((b1 - rmean) * scale + beta)[:, None]
    w2t = W2.T * (1.0 / VOXEL_SIZE)
    b2t = (b2 * (1.0 / VOXEL_SIZE))[:, None]

    pw = jnp.asarray([float(1 << (K - 1 - j)) for j in range(K)],
                     jnp.float32)[:, None]

    grid = (N + BLOCK - 1) // BLOCK
    out_shape = (
        jax.ShapeDtypeStruct((K, N), jnp.float32),
        jax.ShapeDtypeStruct((3, N), jnp.float32),
        jax.ShapeDtypeStruct((1, N), jnp.int32),
        jax.ShapeDtypeStruct((K, 1), jnp.float32),
    )
    logit_t, center_t, seg_t, mean2d = pl.pallas_call(
        _fused_kernel,
        grid=(grid,),
        in_specs=[
            pl.BlockSpec((C, BLOCK), lambda i: (0, i)),
            pl.BlockSpec((3, BLOCK), lambda i: (0, i)),
            pl.BlockSpec((C, C), lambda i: (0, 0)),
            pl.BlockSpec((C, 1), lambda i: (0, 0)),
            pl.BlockSpec((3, C), lambda i: (0, 0)),
            pl.BlockSpec((3, 1), lambda i: (0, 0)),
            pl.BlockSpec((K, C), lambda i: (0, 0)),
            pl.BlockSpec((K, 1), lambda i: (0, 0)),
            pl.BlockSpec((K, 1), lambda i: (0, 0)),
        ],
        out_specs=[
            pl.BlockSpec((K, BLOCK), lambda i: (0, i)),
            pl.BlockSpec((3, BLOCK), lambda i: (0, i)),
            pl.BlockSpec((1, BLOCK), lambda i: (0, i)),
            pl.BlockSpec((K, 1), lambda i: (0, 0)),
        ],
        out_shape=out_shape,
        scratch_shapes=[
            pltpu.VMEM((K, 1), jnp.float32),
            pltpu.VMEM((K, 1), jnp.float32),
        ],
        compiler_params=pltpu.CompilerParams(
            dimension_semantics=("arbitrary",),
        ),
    )(feat.T, coord.T, w1t, b1t, w2t, b2t, Wseg.T, bseg[:, None], pw)

    return logit_t.T, center_t.T, seg_t[0], mean2d[:, 0]


# speedup vs baseline: 1.1411x; 1.0146x over previous
import jax
import jax.numpy as jnp
from jax.experimental import pallas as pl
from jax.experimental.pallas import tpu as pltpu

N = 200000
C = 64
K = 20
VOXEL_SIZE = 0.02
BLOCK = 20480


def _fused_kernel(feat_ref, coord_ref, w1_ref, b1_ref, w2_ref, b2_ref,
                  wseg_ref, bseg_ref, pow_ref,
                  logit_ref, center_ref, seg_ref, mean_ref,
                  s_acc, c_acc):
    i = pl.program_id(0)

    feat = feat_ref[...]

    logits = jnp.dot(wseg_ref[...], feat, preferred_element_type=jnp.float32)
    logits = logits + bseg_ref[...]
    logit_ref[...] = logits

    h = jnp.dot(w1_ref[...], feat, preferred_element_type=jnp.float32)
    h = jnp.maximum(h + b1_ref[...], 0.0)
    bias = jnp.dot(w2_ref[...], h, preferred_element_type=jnp.float32)
    center_ref[...] = coord_ref[...] * (1.0 / VOXEL_SIZE) + (bias + b2_ref[...])

    colmax = jnp.max(logits, axis=0, keepdims=True)
    is_max = (logits == colmax).astype(jnp.float32)
    exps = jnp.exp(logits - colmax)
    packed = jnp.sum(is_max * pow_ref[...], axis=0, keepdims=True)
    expo = (jax.lax.bitcast_convert_type(packed, jnp.int32) >> 23) - 127
    idx = (K - 1) - expo
    seg_ref[...] = idx

    conf = 1.0 / jnp.sum(exps, axis=0, keepdims=True)
    col = jax.lax.broadcasted_iota(jnp.int32, idx.shape, 1) + i * BLOCK
    maskb = jnp.logical_and(idx >= 2, col < N)
    w = jnp.where(maskb, conf, 0.0)
    cnt = jnp.where(maskb, 1.0, 0.0)
    sub = jax.lax.broadcasted_iota(jnp.int32, logits.shape, 0)
    onehot = (sub == idx).astype(jnp.float32)
    scores = jnp.sum(onehot * w, axis=1, keepdims=True)
    counts = jnp.sum(onehot * cnt, axis=1, keepdims=True)

    prev_s = jnp.where(i == 0, jnp.zeros_like(scores), s_acc[...])
    prev_c = jnp.where(i == 0, jnp.zeros_like(counts), c_acc[...])
    s = prev_s + scores
    c = prev_c + counts
    s_acc[...] = s
    c_acc[...] = c
    mean_ref[...] = s / (c + 1e-8)


@jax.jit
def kernel(feat, coord, W1, b1, gamma, beta, rmean, rvar, W2, b2, Wseg, bseg):
    scale = gamma * jax.lax.rsqrt(rvar + 1e-3)
    w1t = W1.T * scale[:, None]
    b1t = ((b1 - rmean) * scale + beta)[:, None]
    w2t = W2.T * (1.0 / VOXEL_SIZE)
    b2t = (b2 * (1.0 / VOXEL_SIZE))[:, None]

    pw = jnp.asarray([float(1 << (K - 1 - j)) for j in range(K)],
                     jnp.float32)[:, None]

    grid = (N + BLOCK - 1) // BLOCK
    out_shape = (
        jax.ShapeDtypeStruct((K, N), jnp.float32),
        jax.ShapeDtypeStruct((3, N), jnp.float32),
        jax.ShapeDtypeStruct((1, N), jnp.int32),
        jax.ShapeDtypeStruct((K, 1), jnp.float32),
    )
    logit_t, center_t, seg_t, mean2d = pl.pallas_call(
        _fused_kernel,
        grid=(grid,),
        in_specs=[
            pl.BlockSpec((C, BLOCK), lambda i: (0, i)),
            pl.BlockSpec((3, BLOCK), lambda i: (0, i)),
            pl.BlockSpec((C, C), lambda i: (0, 0)),
            pl.BlockSpec((C, 1), lambda i: (0, 0)),
            pl.BlockSpec((3, C), lambda i: (0, 0)),
            pl.BlockSpec((3, 1), lambda i: (0, 0)),
            pl.BlockSpec((K, C), lambda i: (0, 0)),
            pl.BlockSpec((K, 1), lambda i: (0, 0)),
            pl.BlockSpec((K, 1), lambda i: (0, 0)),
        ],
        out_specs=[
            pl.BlockSpec((K, BLOCK), lambda i: (0, i)),
            pl.BlockSpec((3, BLOCK), lambda i: (0, i)),
            pl.BlockSpec((1, BLOCK), lambda i: (0, i)),
            pl.BlockSpec((K, 1), lambda i: (0, 0)),
        ],
        out_shape=out_shape,
        scratch_shapes=[
            pltpu.VMEM((K, 1), jnp.float32),
            pltpu.VMEM((K, 1), jnp.float32),
        ],
        compiler_params=pltpu.CompilerParams(
            dimension_semantics=("arbitrary",),
        ),
    )(feat.T, coord.T, w1t, b1t, w2t, b2t, Wseg.T, bseg[:, None], pw)

    return logit_t.T, center_t.T, seg_t[0], mean2d[:, 0]


# speedup vs baseline: 1.3954x; 1.2228x over previous
import jax
import jax.numpy as jnp
from jax.experimental import pallas as pl
from jax.experimental.pallas import tpu as pltpu

N = 200000
C = 64
K = 20
VOXEL_SIZE = 0.02
BLOCK = 20480


def _fused_kernel(feat_ref, coord_ref, w1_ref, b1_ref, w2_ref, b2_ref,
                  wseg_ref, bseg_ref, pow_ref,
                  logit_ref, center_ref, seg_ref, mean_ref,
                  s_acc, c_acc):
    i = pl.program_id(0)

    feat = feat_ref[...]

    logits = jnp.dot(wseg_ref[...], feat, preferred_element_type=jnp.float32)
    logits = logits + bseg_ref[...]
    logit_ref[...] = logits

    h = jnp.dot(w1_ref[...], feat, preferred_element_type=jnp.float32)
    h = jnp.maximum(h + b1_ref[...], 0.0)
    bias = jnp.dot(w2_ref[...], h, preferred_element_type=jnp.float32)
    center_ref[...] = coord_ref[...] * (1.0 / VOXEL_SIZE) + (bias + b2_ref[...])

    colmax = jnp.max(logits, axis=0, keepdims=True)
    is_max = (logits == colmax).astype(jnp.float32)
    exps = jnp.exp(logits - colmax)
    packed = jnp.sum(is_max * pow_ref[...], axis=0, keepdims=True)
    expo = (jax.lax.bitcast_convert_type(packed, jnp.int32) >> 23) - 127
    idx = (K - 1) - expo
    seg_ref[...] = idx.reshape(-1)

    conf = 1.0 / jnp.sum(exps, axis=0, keepdims=True)
    col = jax.lax.broadcasted_iota(jnp.int32, idx.shape, 1) + i * BLOCK
    maskb = jnp.logical_and(idx >= 2, col < N)
    w = jnp.where(maskb, conf, 0.0)
    cnt = jnp.where(maskb, 1.0, 0.0)
    sub = jax.lax.broadcasted_iota(jnp.int32, logits.shape, 0)
    onehot = (sub == idx).astype(jnp.float32)
    scores = jnp.sum(onehot * w, axis=1, keepdims=True)
    counts = jnp.sum(onehot * cnt, axis=1, keepdims=True)

    prev_s = jnp.where(i == 0, jnp.zeros_like(scores), s_acc[...])
    prev_c = jnp.where(i == 0, jnp.zeros_like(counts), c_acc[...])
    s = prev_s + scores
    c = prev_c + counts
    s_acc[...] = s
    c_acc[...] = c
    mean_ref[...] = s / (c + 1e-8)


@jax.jit
def kernel(feat, coord, W1, b1, gamma, beta, rmean, rvar, W2, b2, Wseg, bseg):
    scale = gamma * jax.lax.rsqrt(rvar + 1e-3)
    w1t = W1.T * scale[:, None]
    b1t = ((b1 - rmean) * scale + beta)[:, None]
    w2t = W2.T * (1.0 / VOXEL_SIZE)
    b2t = (b2 * (1.0 / VOXEL_SIZE))[:, None]

    pw = jnp.asarray([float(1 << (K - 1 - j)) for j in range(K)],
                     jnp.float32)[:, None]

    grid = (N + BLOCK - 1) // BLOCK
    out_shape = (
        jax.ShapeDtypeStruct((K, N), jnp.float32),
        jax.ShapeDtypeStruct((3, N), jnp.float32),
        jax.ShapeDtypeStruct((N,), jnp.int32),
        jax.ShapeDtypeStruct((K, 1), jnp.float32),
    )
    logit_t, center_t, seg_t, mean2d = pl.pallas_call(
        _fused_kernel,
        grid=(grid,),
        in_specs=[
            pl.BlockSpec((C, BLOCK), lambda i: (0, i)),
            pl.BlockSpec((3, BLOCK), lambda i: (0, i)),
            pl.BlockSpec((C, C), lambda i: (0, 0)),
            pl.BlockSpec((C, 1), lambda i: (0, 0)),
            pl.BlockSpec((3, C), lambda i: (0, 0)),
            pl.BlockSpec((3, 1), lambda i: (0, 0)),
            pl.BlockSpec((K, C), lambda i: (0, 0)),
            pl.BlockSpec((K, 1), lambda i: (0, 0)),
            pl.BlockSpec((K, 1), lambda i: (0, 0)),
        ],
        out_specs=[
            pl.BlockSpec((K, BLOCK), lambda i: (0, i)),
            pl.BlockSpec((3, BLOCK), lambda i: (0, i)),
            pl.BlockSpec((BLOCK,), lambda i: (i,)),
            pl.BlockSpec((K, 1), lambda i: (0, 0)),
        ],
        out_shape=out_shape,
        scratch_shapes=[
            pltpu.VMEM((K, 1), jnp.float32),
            pltpu.VMEM((K, 1), jnp.float32),
        ],
        compiler_params=pltpu.CompilerParams(
            dimension_semantics=("arbitrary",),
        ),
    )(feat.T, coord.T, w1t, b1t, w2t, b2t, Wseg.T, bseg[:, None], pw)

    return logit_t.T, center_t.T, seg_t, mean2d[:, 0]
